# baseline (speedup 1.0000x reference)
import jax
import jax.numpy as jnp
from jax.experimental import pallas as pl
from jax.experimental.pallas import tpu as pltpu

ALPHA = 2.0
INTENSITY_SHIFT = 10.0

_B, _H, _W = 16, 512, 512
_N_TOTAL = float(_B * _H * _W)


def _body(fg_hbm, bg_hbm, m_ref, out_ref, fg_v, bg_v, m_s, sums_ref,
          fg_sems, bg_sems):
    p = pl.program_id(0)
    i = pl.program_id(1)

    def _chunk_copy(src, dst, sems, j):
        return pltpu.make_async_copy(
            src.at[pl.ds(j, 1)], dst.at[pl.ds(j, 1)], sems.at[j])

    @pl.when((p == 0) & (i == 0))
    def _enqueue_all():
        for j in range(_B):
            _chunk_copy(fg_hbm, fg_v, fg_sems, j).start()
            _chunk_copy(bg_hbm, bg_v, bg_sems, j).start()

    @pl.when(p == 0)
    def _phase0():
        _chunk_copy(fg_hbm, fg_v, fg_sems, i).wait()
        _chunk_copy(bg_hbm, bg_v, bg_sems, i).wait()
        fg = fg_v[pl.ds(i, 1)]
        bg = bg_v[pl.ds(i, 1)]
        m = m_ref[...]
        mf = m.astype(jnp.float32)
        m_s[pl.ds(i, 1)] = m.astype(jnp.int8)
        fgm = fg * mf
        s0 = jnp.sum(bg)
        s1 = jnp.sum(bg * bg)
        s2 = jnp.sum(mf)
        s3 = jnp.sum(fgm)
        s4 = jnp.sum(fg * fgm)
        first = i == 0
        sums_ref[0] = jnp.where(first, s0, sums_ref[0] + s0)
        sums_ref[1] = jnp.where(first, s1, sums_ref[1] + s1)
        sums_ref[2] = jnp.where(first, s2, sums_ref[2] + s2)
        sums_ref[3] = jnp.where(first, s3, sums_ref[3] + s3)
        sums_ref[4] = jnp.where(first, s4, sums_ref[4] + s4)

    @pl.when(p == 1)
    def _phase1():
        s_bg = sums_ref[0]
        s_bg2 = sums_ref[1]
        n = sums_ref[2]
        s_fg = sums_ref[3]
        s_fg2 = sums_ref[4]
        bg_mean = s_bg / _N_TOTAL
        bg_var = (s_bg2 - s_bg * bg_mean) / (_N_TOTAL - 1.0)
        inv_bg_std = jax.lax.rsqrt(bg_var)
        fg_mean = s_fg / n
        fg_var = (s_fg2 - s_fg * fg_mean) / (n - 1.0)
        inv_fg_std = jax.lax.rsqrt(fg_var)
        cf = ALPHA * inv_fg_std
        cc = ALPHA * (INTENSITY_SHIFT - fg_mean * inv_fg_std)
        fg = fg_v[pl.ds(i, 1)]
        bg = bg_v[pl.ds(i, 1)]
        m = m_s[pl.ds(i, 1)] != 0
        bg_std = (bg - bg_mean) * inv_bg_std
        out_ref[...] = jnp.where(m, fg * cf + cc - bg_std, bg_std)


def kernel(foreground, background, mask):
    blk = (1, _H, _W)
    m_spec = pl.BlockSpec(blk, lambda p, i: ((1 - p) * i + p * (_B - 1), 0, 0))
    out_spec = pl.BlockSpec(blk, lambda p, i: (p * i, 0, 0))
    return pl.pallas_call(
        _body,
        grid=(2, _B),
        in_specs=[
            pl.BlockSpec(memory_space=pl.ANY),
            pl.BlockSpec(memory_space=pl.ANY),
            m_spec,
        ],
        out_specs=out_spec,
        out_shape=jax.ShapeDtypeStruct((_B, _H, _W), jnp.float32),
        scratch_shapes=[
            pltpu.VMEM((_B, _H, _W), jnp.float32),
            pltpu.VMEM((_B, _H, _W), jnp.float32),
            pltpu.VMEM((_B, _H, _W), jnp.int8),
            pltpu.SMEM((8,), jnp.float32),
            pltpu.SemaphoreType.DMA((_B,)),
            pltpu.SemaphoreType.DMA((_B,)),
        ],
    )(foreground, background, mask)


# speedup vs baseline: 1.4968x; 1.4968x over previous
import jax
import jax.numpy as jnp
from jax.experimental import pallas as pl
from jax.experimental.pallas import tpu as pltpu

ALPHA = 2.0
INTENSITY_SHIFT = 10.0

_B, _H, _W = 16, 512, 512
_N_TOTAL = float(_B * _H * _W)


def _body(fg_hbm, bg_hbm, m_hbm, out_hbm, fg_v, bg_v, m_v, out_v, sums_ref,
          fg_sems, bg_sems, m_sems, out_sems):
    p = pl.program_id(0)
    i = pl.program_id(1)

    def _cp(src, dst, sems, j):
        return pltpu.make_async_copy(
            src.at[pl.ds(j, 1)], dst.at[pl.ds(j, 1)], sems.at[j])

    @pl.when((p == 0) & (i == 0))
    def _enqueue_all():
        for j in range(_B):
            _cp(fg_hbm, fg_v, fg_sems, j).start()
            _cp(bg_hbm, bg_v, bg_sems, j).start()
            _cp(m_hbm, m_v, m_sems, j).start()

    @pl.when(p == 0)
    def _phase0():
        _cp(fg_hbm, fg_v, fg_sems, i).wait()
        _cp(bg_hbm, bg_v, bg_sems, i).wait()
        _cp(m_hbm, m_v, m_sems, i).wait()
        fg = fg_v[pl.ds(i, 1)]
        bg = bg_v[pl.ds(i, 1)]
        mf = (m_v[pl.ds(i, 1)] != 0).astype(jnp.float32)
        fgm = fg * mf
        s0 = jnp.sum(bg)
        s1 = jnp.sum(bg * bg)
        s2 = jnp.sum(mf)
        s3 = jnp.sum(fgm)
        s4 = jnp.sum(fgm * fg)
        first = i == 0
        sums_ref[0] = jnp.where(first, s0, sums_ref[0] + s0)
        sums_ref[1] = jnp.where(first, s1, sums_ref[1] + s1)
        sums_ref[2] = jnp.where(first, s2, sums_ref[2] + s2)
        sums_ref[3] = jnp.where(first, s3, sums_ref[3] + s3)
        sums_ref[4] = jnp.where(first, s4, sums_ref[4] + s4)

    @pl.when(p == 1)
    def _phase1():
        s_bg = sums_ref[0]
        s_bg2 = sums_ref[1]
        n = sums_ref[2]
        s_fg = sums_ref[3]
        s_fg2 = sums_ref[4]
        bg_mean = s_bg / _N_TOTAL
        bg_var = (s_bg2 - s_bg * bg_mean) / (_N_TOTAL - 1.0)
        inv_bg_std = jax.lax.rsqrt(bg_var)
        fg_mean = s_fg / n
        fg_var = (s_fg2 - s_fg * fg_mean) / (n - 1.0)
        inv_fg_std = jax.lax.rsqrt(fg_var)
        cf = ALPHA * inv_fg_std
        cc = ALPHA * (INTENSITY_SHIFT - fg_mean * inv_fg_std)
        fg = fg_v[pl.ds(i, 1)]
        bg = bg_v[pl.ds(i, 1)]
        m = m_v[pl.ds(i, 1)] != 0
        bg_std = (bg - bg_mean) * inv_bg_std
        out_v[pl.ds(i, 1)] = jnp.where(m, fg * cf + cc - bg_std, bg_std)
        _cp(out_v, out_hbm, out_sems, i).start()

        @pl.when(i == _B - 1)
        def _drain():
            for j in range(_B):
                _cp(out_v, out_hbm, out_sems, j).wait()


def kernel(foreground, background, mask):
    mask8 = mask.view(jnp.int8)
    return pl.pallas_call(
        _body,
        grid=(2, _B),
        in_specs=[
            pl.BlockSpec(memory_space=pl.ANY),
            pl.BlockSpec(memory_space=pl.ANY),
            pl.BlockSpec(memory_space=pl.ANY),
        ],
        out_specs=pl.BlockSpec(memory_space=pl.ANY),
        out_shape=jax.ShapeDtypeStruct((_B, _H, _W), jnp.float32),
        scratch_shapes=[
            pltpu.VMEM((_B, _H, _W), jnp.float32),
            pltpu.VMEM((_B, _H, _W), jnp.float32),
            pltpu.VMEM((_B, _H, _W), jnp.int8),
            pltpu.VMEM((_B, _H, _W), jnp.float32),
            pltpu.SMEM((8,), jnp.float32),
            pltpu.SemaphoreType.DMA((_B,)),
            pltpu.SemaphoreType.DMA((_B,)),
            pltpu.SemaphoreType.DMA((_B,)),
            pltpu.SemaphoreType.DMA((_B,)),
        ],
    )(foreground, background, mask8)
